# baseline (speedup 1.0000x reference)
import jax
import jax.numpy as jnp
from jax.experimental import pallas as pl
from jax.experimental.pallas import tpu as pltpu

_HI = jax.lax.Precision.HIGHEST


def _dot(a, b):
    return jax.lax.dot_general(
        a, b, (((1,), (0,)), ((), ())),
        precision=_HI, preferred_element_type=jnp.float32)


def _dotbf(a, b):
    return jax.lax.dot_general(
        a.astype(jnp.bfloat16), b.astype(jnp.bfloat16), (((1,), (0,)), ((), ())),
        preferred_element_type=jnp.float32)


def _leaky(x):
    return jnp.where(x >= 0, x, 0.1 * x)


def _body(x_ref, w1, b1, w2, b2, w3, b3, w4, b4, embd, embd_t,
          w5, b5, w6, b6, w7, b7, w8, b8,
          xrec_ref, zenc_ref, zdec_ref, zfe_ref, runmin_ref):
    i = pl.program_id(0)
    blk = x_ref.shape[0]
    K, D = embd.shape

    h = jnp.maximum(_dotbf(x_ref[...], w1[...]) + b1[...], 0.0)
    h = jnp.maximum(_dotbf(h, w2[...]) + b2[...], 0.0)
    h = jnp.maximum(_dotbf(h, w3[...]) + b3[...], 0.0)
    z = _dotbf(h, w4[...]) + b4[...]
    zenc_ref[...] = z

    qsq = jnp.sum(z * z, axis=1, keepdims=True)
    tsq = jnp.sum(embd[...] * embd[...], axis=1)
    g = _dotbf(z, embd_t[...])
    d2 = jnp.maximum(qsq + tsq[None, :] - 2.0 * g, 0.0)

    iota_k = jax.lax.broadcasted_iota(jnp.int32, (blk, K), 1)
    dmin = jnp.min(d2, axis=1, keepdims=True)
    idx = jnp.min(jnp.where(d2 == dmin, iota_k, K), axis=1, keepdims=True)
    onehot = (iota_k == idx).astype(jnp.float32)
    zq = _dot(onehot, embd[...])
    zdec_ref[...] = zq

    iota_r = jax.lax.broadcasted_iota(jnp.int32, (blk, K), 0)
    bmin = jnp.min(d2, axis=0)
    brow = jnp.min(jnp.where(d2 == bmin[None, :], iota_r, blk), axis=0)

    @pl.when(i == 0)
    def _():
        runmin_ref[...] = jnp.full(runmin_ref.shape, jnp.inf, jnp.float32)

    bmin_c = bmin.reshape(K, 1)
    brow_c = brow.reshape(K, 1)
    better = bmin_c < runmin_ref[...]
    runmin_ref[...] = jnp.where(better, bmin_c, runmin_ref[...])
    iota_b = jax.lax.broadcasted_iota(jnp.int32, (K, blk), 1)
    onehot2 = (iota_b == brow_c).astype(jnp.float32)
    rows = _dot(onehot2, z)
    zfe_ref[...] = jnp.where(better, rows, zfe_ref[...])

    d = _leaky(_dotbf(zq, w5[...]) + b5[...])
    d = _leaky(_dotbf(d, w6[...]) + b6[...])
    d = _leaky(_dotbf(d, w7[...]) + b7[...])
    xrec_ref[...] = jnp.tanh(_dotbf(d, w8[...]) + b8[...])


def kernel(X, W1, b1, W2, b2, W3, b3, W4, b4, embd, W5, b5, W6, b6, W7, b7, W8, b8):
    B = X.shape[0]
    Xf = X.reshape(B, 784)
    K, D = embd.shape
    BLK = 512
    nblk = B // BLK

    def full(arr):
        return pl.BlockSpec(arr.shape, lambda i: (0,) * arr.ndim)

    row2 = lambda n: pl.BlockSpec((BLK, n), lambda i: (i, 0))
    biases = [b.reshape(1, -1) for b in (b1, b2, b3, b4, b5, b6, b7, b8)]
    bf = lambda w: w.astype(jnp.bfloat16)
    weights = (bf(W1), biases[0], bf(W2), biases[1], bf(W3), biases[2],
               bf(W4), biases[3], embd, bf(embd.T), bf(W5), biases[4],
               bf(W6), biases[5], bf(W7), biases[6], bf(W8), biases[7])

    out = pl.pallas_call(
        _body,
        grid=(nblk,),
        in_specs=[row2(784)] + [full(w) for w in weights],
        out_specs=[row2(784), row2(D), row2(D),
                   pl.BlockSpec((K, D), lambda i: (0, 0))],
        out_shape=[
            jax.ShapeDtypeStruct((B, 784), jnp.float32),
            jax.ShapeDtypeStruct((B, D), jnp.float32),
            jax.ShapeDtypeStruct((B, D), jnp.float32),
            jax.ShapeDtypeStruct((K, D), jnp.float32),
        ],
        scratch_shapes=[pltpu.VMEM((K, 1), jnp.float32)],
        compiler_params=pltpu.CompilerParams(
            dimension_semantics=("arbitrary",)),
    )(Xf, *weights)

    X_recon, Z_enc, Z_dec, Zfe = out
    return (X_recon.reshape(B, 1, 28, 28), Z_enc, Z_dec, Zfe)


# speedup vs baseline: 1.1134x; 1.1134x over previous
import jax
import jax.numpy as jnp
from jax.experimental import pallas as pl
from jax.experimental.pallas import tpu as pltpu

_HI = jax.lax.Precision.HIGHEST


def _dot(a, b):
    return jax.lax.dot_general(
        a, b, (((1,), (0,)), ((), ())),
        precision=_HI, preferred_element_type=jnp.float32)


def _dotbf(a, b):
    return jax.lax.dot_general(
        a.astype(jnp.bfloat16), b.astype(jnp.bfloat16), (((1,), (0,)), ((), ())),
        preferred_element_type=jnp.float32)


def _leaky(x):
    return jnp.where(x >= 0, x, 0.1 * x)


def _body(x_ref, w1, b1, w2, b2, w3, b3, w4, b4, embd, embd_hi, embd_lo,
          embd_t, w5, b5, w6, b6, w7, b7, w8, b8,
          xrec_ref, zenc_ref, zdec_ref, zfe_ref, runmin_ref):
    i = pl.program_id(0)
    blk = x_ref.shape[0]
    K, D = embd.shape

    h = jnp.maximum(_dotbf(x_ref[...], w1[...]) + b1[...], 0.0)
    h = jnp.maximum(_dotbf(h, w2[...]) + b2[...], 0.0)
    h = jnp.maximum(_dotbf(h, w3[...]) + b3[...], 0.0)
    z = _dotbf(h, w4[...]) + b4[...]
    zenc_ref[...] = z

    qsq = jnp.sum(z * z, axis=1, keepdims=True)
    tsq = jnp.sum(embd[...] * embd[...], axis=1)
    g = _dotbf(z, embd_t[...])
    d2 = jnp.maximum(qsq + tsq[None, :] - 2.0 * g, 0.0)

    iota_k = jax.lax.broadcasted_iota(jnp.int32, (blk, K), 1)
    dmin = jnp.min(d2, axis=1, keepdims=True)
    idx = jnp.min(jnp.where(d2 == dmin, iota_k, K), axis=1, keepdims=True)
    onehot = (iota_k == idx).astype(jnp.bfloat16)
    zq = _dotbf(onehot, embd_hi[...]) + _dotbf(onehot, embd_lo[...])
    zdec_ref[...] = zq

    iota_r = jax.lax.broadcasted_iota(jnp.int32, (blk, K), 0)
    bmin = jnp.min(d2, axis=0)
    brow = jnp.min(jnp.where(d2 == bmin[None, :], iota_r, blk), axis=0)

    @pl.when(i == 0)
    def _():
        runmin_ref[...] = jnp.full(runmin_ref.shape, jnp.inf, jnp.float32)

    bmin_c = bmin.reshape(K, 1)
    brow_c = brow.reshape(K, 1)
    better = bmin_c < runmin_ref[...]
    runmin_ref[...] = jnp.where(better, bmin_c, runmin_ref[...])
    iota_b = jax.lax.broadcasted_iota(jnp.int32, (K, blk), 1)
    onehot2 = (iota_b == brow_c).astype(jnp.bfloat16)
    z_hi = z.astype(jnp.bfloat16)
    z_lo = (z - z_hi.astype(jnp.float32)).astype(jnp.bfloat16)
    rows = _dotbf(onehot2, z_hi) + _dotbf(onehot2, z_lo)
    zfe_ref[...] = jnp.where(better, rows, zfe_ref[...])

    d = _leaky(_dotbf(zq, w5[...]) + b5[...])
    d = _leaky(_dotbf(d, w6[...]) + b6[...])
    d = _leaky(_dotbf(d, w7[...]) + b7[...])
    xrec_ref[...] = jnp.tanh(_dotbf(d, w8[...]) + b8[...])


def kernel(X, W1, b1, W2, b2, W3, b3, W4, b4, embd, W5, b5, W6, b6, W7, b7, W8, b8):
    B = X.shape[0]
    Xf = X.reshape(B, 784)
    K, D = embd.shape
    BLK = 512
    nblk = B // BLK

    def full(arr):
        return pl.BlockSpec(arr.shape, lambda i: (0,) * arr.ndim)

    row2 = lambda n: pl.BlockSpec((BLK, n), lambda i: (i, 0))
    biases = [b.reshape(1, -1) for b in (b1, b2, b3, b4, b5, b6, b7, b8)]
    bf = lambda w: w.astype(jnp.bfloat16)
    embd_hi = bf(embd)
    embd_lo = bf(embd - embd_hi.astype(jnp.float32))
    weights = (bf(W1), biases[0], bf(W2), biases[1], bf(W3), biases[2],
               bf(W4), biases[3], embd, embd_hi, embd_lo, bf(embd.T),
               bf(W5), biases[4], bf(W6), biases[5], bf(W7), biases[6],
               bf(W8), biases[7])

    out = pl.pallas_call(
        _body,
        grid=(nblk,),
        in_specs=[row2(784)] + [full(w) for w in weights],
        out_specs=[row2(784), row2(D), row2(D),
                   pl.BlockSpec((K, D), lambda i: (0, 0))],
        out_shape=[
            jax.ShapeDtypeStruct((B, 784), jnp.float32),
            jax.ShapeDtypeStruct((B, D), jnp.float32),
            jax.ShapeDtypeStruct((B, D), jnp.float32),
            jax.ShapeDtypeStruct((K, D), jnp.float32),
        ],
        scratch_shapes=[pltpu.VMEM((K, 1), jnp.float32)],
        compiler_params=pltpu.CompilerParams(
            dimension_semantics=("arbitrary",)),
    )(Xf, *weights)

    X_recon, Z_enc, Z_dec, Zfe = out
    return (X_recon.reshape(B, 1, 28, 28), Z_enc, Z_dec, Zfe)


# speedup vs baseline: 1.4173x; 1.2729x over previous
import jax
import jax.numpy as jnp
from jax.experimental import pallas as pl
from jax.experimental.pallas import tpu as pltpu

_HI = jax.lax.Precision.HIGHEST


def _dot(a, b):
    return jax.lax.dot_general(
        a, b, (((1,), (0,)), ((), ())),
        precision=_HI, preferred_element_type=jnp.float32)


def _dotbf(a, b):
    return jax.lax.dot_general(
        a.astype(jnp.bfloat16), b.astype(jnp.bfloat16), (((1,), (0,)), ((), ())),
        preferred_element_type=jnp.float32)


def _leaky(x):
    return jnp.where(x >= 0, x, 0.1 * x)


def _body(x_ref, w1, b1, w2, b2, w3, b3, w4, b4, embd, embd_hi, embd_lo,
          embd_t, w5, b5, w6, b6, w7, b7, w8, b8,
          xrec_ref, zenc_ref, zdec_ref, zfe_ref, runmin_ref):
    i = pl.program_id(0)
    blk = x_ref.shape[0]
    K, D = embd.shape

    x = x_ref[...].reshape(blk, 784)
    h = jnp.maximum(_dotbf(x, w1[...]) + b1[...], 0.0)
    h = jnp.maximum(_dotbf(h, w2[...]) + b2[...], 0.0)
    h = jnp.maximum(_dotbf(h, w3[...]) + b3[...], 0.0)
    z = _dotbf(h, w4[...]) + b4[...]
    zenc_ref[...] = z

    qsq = jnp.sum(z * z, axis=1, keepdims=True)
    tsq = jnp.sum(embd[...] * embd[...], axis=1)
    g = _dotbf(z, embd_t[...])
    d2 = jnp.maximum(qsq + tsq[None, :] - 2.0 * g, 0.0)

    iota_k = jax.lax.broadcasted_iota(jnp.int32, (blk, K), 1)
    dmin = jnp.min(d2, axis=1, keepdims=True)
    idx = jnp.min(jnp.where(d2 == dmin, iota_k, K), axis=1, keepdims=True)
    onehot = (iota_k == idx).astype(jnp.bfloat16)
    zq = _dotbf(onehot, embd_hi[...]) + _dotbf(onehot, embd_lo[...])
    zdec_ref[...] = zq

    iota_r = jax.lax.broadcasted_iota(jnp.int32, (blk, K), 0)
    bmin = jnp.min(d2, axis=0)
    brow = jnp.min(jnp.where(d2 == bmin[None, :], iota_r, blk), axis=0)

    @pl.when(i == 0)
    def _():
        runmin_ref[...] = jnp.full(runmin_ref.shape, jnp.inf, jnp.float32)

    bmin_c = bmin.reshape(K, 1)
    brow_c = brow.reshape(K, 1)
    better = bmin_c < runmin_ref[...]
    runmin_ref[...] = jnp.where(better, bmin_c, runmin_ref[...])
    iota_b = jax.lax.broadcasted_iota(jnp.int32, (K, blk), 1)
    onehot2 = (iota_b == brow_c).astype(jnp.bfloat16)
    z_hi = z.astype(jnp.bfloat16)
    z_lo = (z - z_hi.astype(jnp.float32)).astype(jnp.bfloat16)
    rows = _dotbf(onehot2, z_hi) + _dotbf(onehot2, z_lo)
    zfe_ref[...] = jnp.where(better, rows, zfe_ref[...])

    d = _leaky(_dotbf(zq, w5[...]) + b5[...])
    d = _leaky(_dotbf(d, w6[...]) + b6[...])
    d = _leaky(_dotbf(d, w7[...]) + b7[...])
    xr = jnp.tanh(_dotbf(d, w8[...]) + b8[...])
    xrec_ref[...] = xr.reshape(xrec_ref.shape)


def kernel(X, W1, b1, W2, b2, W3, b3, W4, b4, embd, W5, b5, W6, b6, W7, b7, W8, b8):
    B = X.shape[0]
    K, D = embd.shape
    BLK = 512
    nblk = B // BLK

    def full(arr):
        return pl.BlockSpec(arr.shape, lambda i: (0,) * arr.ndim)

    row2 = lambda n: pl.BlockSpec((BLK, n), lambda i: (i, 0))
    row4 = pl.BlockSpec((BLK, 1, 28, 28), lambda i: (i, 0, 0, 0))
    biases = [b.reshape(1, -1) for b in (b1, b2, b3, b4, b5, b6, b7, b8)]
    bf = lambda w: w.astype(jnp.bfloat16)
    embd_hi = bf(embd)
    embd_lo = bf(embd - embd_hi.astype(jnp.float32))
    weights = (bf(W1), biases[0], bf(W2), biases[1], bf(W3), biases[2],
               bf(W4), biases[3], embd, embd_hi, embd_lo, bf(embd.T),
               bf(W5), biases[4], bf(W6), biases[5], bf(W7), biases[6],
               bf(W8), biases[7])

    out = pl.pallas_call(
        _body,
        grid=(nblk,),
        in_specs=[row4] + [full(w) for w in weights],
        out_specs=[row4, row2(D), row2(D),
                   pl.BlockSpec((K, D), lambda i: (0, 0))],
        out_shape=[
            jax.ShapeDtypeStruct((B, 1, 28, 28), jnp.float32),
            jax.ShapeDtypeStruct((B, D), jnp.float32),
            jax.ShapeDtypeStruct((B, D), jnp.float32),
            jax.ShapeDtypeStruct((K, D), jnp.float32),
        ],
        scratch_shapes=[pltpu.VMEM((K, 1), jnp.float32)],
        compiler_params=pltpu.CompilerParams(
            dimension_semantics=("arbitrary",)),
    )(X, *weights)

    X_recon, Z_enc, Z_dec, Zfe = out
    return (X_recon, Z_enc, Z_dec, Zfe)
